# Initial kernel scaffold; baseline (speedup 1.0000x reference)
import jax
import jax.numpy as jnp
from jax.experimental import pallas as pl


def kernel(x, edge_index, edge_attr, W1, b1, g1, beta1, W2, b2, g2, beta2, Wg, bg, Wf, bf):
    raise NotImplementedError("write your pallas kernel here")


# speedup vs baseline: 4.8860x; 4.8860x over previous
import functools

import jax
import jax.numpy as jnp
from jax import lax
from jax.experimental import pallas as pl
from jax.experimental.pallas import tpu as pltpu
from jax.experimental.pallas import tpu_sc as plsc

N = 10000
E = 320000
D = 128
H = 128
DE = 16
OUT = 256

NC = 2
NS = 16
NW = NC * NS
ESL = E // NS
CH = 64
NP = 10240
HALF = NP // 2
RANGE = HALF // 2
ACCR = RANGE + 256
NPA = N + 48
ZPT = ACCR // NS
OPT = RANGE // NS
OCH = 32
ERB = (ESL + CH) // CH
NV = H // 16

_MESH = plsc.VectorSubcoreMesh(core_axis_name="c", subcore_axis_name="s")


def _zero_rows(buf, nrows, nv):
    z = jnp.zeros((16,), jnp.float32)

    def row(r, carry):
        for v in range(nv):
            buf[r, pl.ds(v * 16, 16)] = z
        return carry

    lax.fori_loop(0, nrows, row, None)


def _scatter_body(mode, *refs):
    with_c = mode != "gcn"
    if mode == "edge":
        (a_hbm, b_hbm, c_hbm, dst_hbm, src_hbm, out_hbm,
         dst_v, src_v, eid_v, std_v, sts_v, ste_v, stg_v, m_v, c_v, cp_v,
         acc) = refs
    else:
        (b_hbm, dst_hbm, src_hbm, out_hbm,
         dst_v, src_v, eid_v, std_v, sts_v, m_v, cp_v, acc) = refs
        a_hbm = c_hbm = ste_v = stg_v = c_v = None

    cid = lax.axis_index("c")
    sid = lax.axis_index("s")
    i16 = lax.iota(jnp.int32, 16)
    sbase = pl.multiple_of(sid * ESL, ESL)

    pltpu.sync_copy(dst_hbm.at[pl.ds(sbase, ESL)], dst_v.at[pl.ds(0, ESL)])
    pltpu.sync_copy(src_hbm.at[pl.ds(sbase, ESL)], src_v.at[pl.ds(0, ESL)])

    for p in range(2):
        lo = cid * HALF + p * RANGE

        _zero_rows(cp_v, 16, NV)
        zslab = pl.multiple_of(sid * ZPT, 16)
        for k in range(ZPT // 16):
            pltpu.sync_copy(cp_v.at[pl.ds(0, 16)],
                            acc.at[pl.ds(zslab + k * 16, 16)])

        def part(i, cnt):
            dv = dst_v[pl.ds(i * 16, 16)]
            m = (dv >= lo) & (dv < lo + RANGE)
            mi = m.astype(jnp.int32)
            pos = cnt + plsc.cumsum(mi) - 1
            ri = lax.shift_right_logical(pos, 6)
            ci = pos & (CH - 1)
            plsc.store_scatter(eid_v, [ri, ci], sbase + i * 16 + i16, mask=m)
            return cnt + jnp.sum(mi)

        cnt = lax.fori_loop(0, ESL // 16, part, jnp.int32(0))
        nch = lax.shift_right_logical(cnt + (CH - 1), 6)
        cnt_pad = nch * CH
        sent = jnp.full((16,), sbase + ESL, jnp.int32)
        for k in range(CH // 16):
            pos = cnt + k * 16 + i16
            m = pos < cnt_pad
            ri = lax.shift_right_logical(pos, 6)
            ci = pos & (CH - 1)
            plsc.store_scatter(eid_v, [ri, ci], sent, mask=m)
        plsc.subcore_barrier()

        def chunk(j, carry):
            for k in range(CH // 16):
                eid = eid_v[j, pl.ds(k * 16, 16)]
                el = eid - sbase
                valid = el < ESL
                elc = jnp.minimum(el, ESL)
                dv = plsc.load_gather(dst_v, [elc])
                sv = plsc.load_gather(src_v, [elc])
                std_v[0, pl.ds(k * 16, 16)] = jnp.where(valid, dv - lo, RANGE)
                sts_v[0, pl.ds(k * 16, 16)] = jnp.where(valid, sv, 0)
                if with_c:
                    ste_v[0, pl.ds(k * 16, 16)] = jnp.where(valid, eid, 0)
                    stg_v[0, pl.ds(k * 16, 16)] = jnp.where(valid, dv, 0)
            pltpu.sync_copy(b_hbm.at[sts_v.at[0]], m_v)
            if with_c:
                pltpu.sync_copy(a_hbm.at[stg_v.at[0]], m_v, add=True)
                pltpu.sync_copy(c_hbm.at[ste_v.at[0]], c_v)

                def row(r, rc):
                    for v in range(NV):
                        s = pl.ds(v * 16, 16)
                        m_v[r, s] = jnp.maximum(m_v[r, s] + c_v[r, s], 0.0)
                    return rc

                lax.fori_loop(0, CH, row, None)
            pltpu.sync_copy(m_v, acc.at[std_v.at[0]], add=True)
            return carry

        lax.fori_loop(0, nch, chunk, None)
        plsc.subcore_barrier()

        gbase = pl.multiple_of(cid * HALF, HALF) + p * RANGE
        oslab = pl.multiple_of(sid * OPT, OCH)
        for k in range(OPT // OCH):
            r0 = oslab + k * OCH
            pltpu.sync_copy(acc.at[pl.ds(r0, OCH)], cp_v)
            pltpu.sync_copy(cp_v, out_hbm.at[pl.ds(gbase + r0, OCH)])
        plsc.subcore_barrier()


def _deg_body(dst_hbm, deg_hbm, dst_v, deg_t):
    cid = lax.axis_index("c")
    sid = lax.axis_index("s")
    sbase = pl.multiple_of(sid * ESL, ESL)
    pltpu.sync_copy(dst_hbm.at[pl.ds(sbase, ESL)], dst_v.at[pl.ds(0, ESL)])
    zf = jnp.zeros((16,), jnp.float32)

    def zdeg(i, carry):
        deg_t[pl.ds(i * 16, 16)] = zf
        return carry

    lax.fori_loop(0, NP // 16, zdeg, None)
    onesf = jnp.ones((16,), jnp.float32)
    dbase = pl.multiple_of(cid * (ESL // 2), ESL // 2)

    def dloop(i, carry):
        dv = dst_v[pl.ds(dbase + i * 16, 16)]
        plsc.addupdate_scatter(deg_t, [dv], onesf)
        return carry

    lax.fori_loop(0, ESL // 2 // 16, dloop, None)
    wid = sid * NC + cid
    pltpu.sync_copy(deg_t, deg_hbm.at[pl.ds(pl.multiple_of(wid * NP, NP), NP)])


_deg_count = pl.kernel(
    _deg_body,
    out_type=(jax.ShapeDtypeStruct((NW * NP,), jnp.float32),),
    mesh=_MESH,
    compiler_params=pltpu.CompilerParams(needs_layout_passes=False),
    scratch_types=[
        pltpu.VMEM((ESL + 16,), jnp.int32),
        pltpu.VMEM((NP,), jnp.float32),
    ],
)


_COMMON_SCRATCH = [
    pltpu.VMEM((ESL + 16,), jnp.int32),
    pltpu.VMEM((ESL + 16,), jnp.int32),
    pltpu.VMEM((ERB, CH), jnp.int32),
    pltpu.VMEM((1, CH), jnp.int32),
    pltpu.VMEM((1, CH), jnp.int32),
]

_edge_scatter = pl.kernel(
    functools.partial(_scatter_body, "edge"),
    out_type=(jax.ShapeDtypeStruct((NP, H), jnp.float32),),
    mesh=_MESH,
    compiler_params=pltpu.CompilerParams(needs_layout_passes=False),
    scratch_types=_COMMON_SCRATCH + [
        pltpu.VMEM((1, CH), jnp.int32),
        pltpu.VMEM((1, CH), jnp.int32),
        pltpu.VMEM((CH, H), jnp.float32),
        pltpu.VMEM((CH, H), jnp.float32),
        pltpu.VMEM((OCH, H), jnp.float32),
        pltpu.VMEM_SHARED((ACCR, H), jnp.float32),
    ],
)

_gcn_scatter = pl.kernel(
    functools.partial(_scatter_body, "gcn"),
    out_type=(jax.ShapeDtypeStruct((NP, H), jnp.float32),),
    mesh=_MESH,
    compiler_params=pltpu.CompilerParams(needs_layout_passes=False),
    scratch_types=_COMMON_SCRATCH + [
        pltpu.VMEM((CH, H), jnp.float32),
        pltpu.VMEM((OCH, H), jnp.float32),
        pltpu.VMEM_SHARED((ACCR, H), jnp.float32),
    ],
)


def _proj_body(x_ref, wa_ref, wb_ref, a_ref, b_ref):
    xv = x_ref[...]
    a_ref[...] = jnp.dot(xv, wa_ref[...], preferred_element_type=jnp.float32)
    b_ref[...] = jnp.dot(xv, wb_ref[...], preferred_element_type=jnp.float32)


_proj = pl.pallas_call(
    _proj_body,
    out_shape=(jax.ShapeDtypeStruct((NPA, H), jnp.float32),
               jax.ShapeDtypeStruct((NPA, H), jnp.float32)),
)

_BE = 2000


def _cfeat_body(ea_ref, w1e_ref, b1_ref, w2e_ref, b2_ref, c1_ref, c2_ref):
    eav = ea_ref[...]
    c1_ref[...] = jnp.dot(eav, w1e_ref[...],
                          preferred_element_type=jnp.float32) + b1_ref[...]
    c2_ref[...] = jnp.dot(eav, w2e_ref[...],
                          preferred_element_type=jnp.float32) + b2_ref[...]


_cfeat = pl.pallas_call(
    _cfeat_body,
    grid=(E // _BE,),
    in_specs=[
        pl.BlockSpec((_BE, DE), lambda i: (i, 0)),
        pl.BlockSpec((DE, H), lambda i: (0, 0)),
        pl.BlockSpec((1, H), lambda i: (0, 0)),
        pl.BlockSpec((DE, H), lambda i: (0, 0)),
        pl.BlockSpec((1, H), lambda i: (0, 0)),
    ],
    out_specs=(pl.BlockSpec((_BE, H), lambda i: (i, 0)),
               pl.BlockSpec((_BE, H), lambda i: (i, 0))),
    out_shape=(jax.ShapeDtypeStruct((E, H), jnp.float32),
               jax.ShapeDtypeStruct((E, H), jnp.float32)),
)


def _bn_relu(h, g_ref, beta_ref):
    m = jnp.mean(h, axis=0, keepdims=True)
    v = jnp.mean((h - m) ** 2, axis=0, keepdims=True)
    hn = (h - m) * lax.rsqrt(v + 1e-5) * g_ref[...] + beta_ref[...]
    return jnp.maximum(hn, 0.0)


def _mid1_body(hp_ref, g1_ref, beta1_ref, w2a_ref, w2b_ref, a2_ref, b2_ref):
    h = hp_ref[:N]
    hn = _bn_relu(h, g1_ref, beta1_ref)
    zpad = jnp.zeros((NPA - N, H), jnp.float32)
    a2_ref[:N] = jnp.dot(hn, w2a_ref[...], preferred_element_type=jnp.float32)
    a2_ref[N:] = zpad
    b2_ref[:N] = jnp.dot(hn, w2b_ref[...], preferred_element_type=jnp.float32)
    b2_ref[N:] = zpad


_mid1 = pl.pallas_call(
    _mid1_body,
    out_shape=(jax.ShapeDtypeStruct((NPA, H), jnp.float32),
               jax.ShapeDtypeStruct((NPA, H), jnp.float32)),
)


def _dinv_from(degp_ref):
    deg = jnp.sum(degp_ref[:, :N], axis=0) + 1.0
    return lax.rsqrt(deg)[:, None]


def _mid2_body(hp_ref, g2_ref, beta2_ref, wg_ref, degp_ref, gd_ref):
    h = hp_ref[:N]
    hn = _bn_relu(h, g2_ref, beta2_ref)
    g = jnp.dot(hn, wg_ref[...], preferred_element_type=jnp.float32)
    gd_ref[...] = g * _dinv_from(degp_ref)


_mid2 = pl.pallas_call(
    _mid2_body,
    out_shape=jax.ShapeDtypeStruct((N, H), jnp.float32),
)


def _final_body(sp_ref, gd_ref, degp_ref, bg_ref, wf_ref, bf_ref, out_ref):
    s = sp_ref[:N] + gd_ref[...]
    hg = s * _dinv_from(degp_ref) + bg_ref[...]
    out_ref[...] = jnp.dot(hg, wf_ref[...],
                           preferred_element_type=jnp.float32) + bf_ref[...]


_final = pl.pallas_call(
    _final_body,
    out_shape=jax.ShapeDtypeStruct((N, OUT), jnp.float32),
)


def kernel(x, edge_index, edge_attr, W1, b1, g1, beta1, W2, b2, g2, beta2,
           Wg, bg, Wf, bf):
    ei = edge_index.astype(jnp.int32)
    src = ei[0]
    dst = ei[1]

    x_pad = jnp.concatenate([x, jnp.zeros((NPA - N, D), jnp.float32)])
    a1, b1p = _proj(x_pad, W1[:D], W1[D:2 * D])
    c1, c2 = _cfeat(edge_attr, W1[2 * D:], b1.reshape(1, H),
                    W2[2 * H:], b2.reshape(1, H))

    (hp,) = _edge_scatter(a1, b1p, c1, dst, src)
    (degf,) = _deg_count(dst)
    degp = degf.reshape(NW, NP)
    a2, b2p = _mid1(hp, g1.reshape(1, H), beta1.reshape(1, H),
                    W2[:H], W2[H:2 * H])
    (hp2,) = _edge_scatter(a2, b2p, c2, dst, src)
    gd = _mid2(hp2, g2.reshape(1, H), beta2.reshape(1, H), Wg, degp)
    (sp,) = _gcn_scatter(gd, dst, src)
    return _final(sp, gd, degp, bg.reshape(1, H), Wf, bf.reshape(1, OUT))


# speedup vs baseline: 5.2747x; 1.0796x over previous
import functools

import jax
import jax.numpy as jnp
from jax import lax
from jax.experimental import pallas as pl
from jax.experimental.pallas import tpu as pltpu
from jax.experimental.pallas import tpu_sc as plsc

N = 10000
E = 320000
D = 128
H = 128
DE = 16
OUT = 256

NC = 2
NS = 16
NW = NC * NS
ESL = E // NS
CH = 64
NP = 10240
HALF = NP // 2
RANGE = HALF // 2
ACCR = RANGE + 8
NPA = N + 48
ZPT = RANGE // NS
OPT = RANGE // NS
OCH = 32
ERB = (ESL + CH) // CH
NV = H // 16

_MESH = plsc.VectorSubcoreMesh(core_axis_name="c", subcore_axis_name="s")


def _zero_rows(buf, nrows, nv):
    z = jnp.zeros((16,), jnp.float32)

    def row(r, carry):
        for v in range(nv):
            buf[r, pl.ds(v * 16, 16)] = z
        return carry

    lax.fori_loop(0, nrows, row, None)


def _scatter_body(mode, *refs):
    with_c = mode != "gcn"
    if mode == "edge":
        (ab_hbm, c_hbm, dst_hbm, src_hbm, out_hbm,
         dst_v, src_v, eid_v, std_v, stab_v, ste_v, ab_v, c_v, cp_v,
         acc) = refs
    else:
        (b_hbm, dst_hbm, src_hbm, out_hbm,
         dst_v, src_v, eid_v, std_v, sts_v, m_v, cp_v, acc) = refs
        c_hbm = ste_v = stab_v = ab_v = c_v = None

    cid = lax.axis_index("c")
    sid = lax.axis_index("s")
    i16 = lax.iota(jnp.int32, 16)
    sbase = pl.multiple_of(sid * ESL, ESL)

    pltpu.sync_copy(dst_hbm.at[pl.ds(sbase, ESL)], dst_v.at[pl.ds(0, ESL)])
    pltpu.sync_copy(src_hbm.at[pl.ds(sbase, ESL)], src_v.at[pl.ds(0, ESL)])

    for p in range(2):
        lo = cid * HALF + p * RANGE

        _zero_rows(cp_v, 16, NV)
        zslab = pl.multiple_of(sid * ZPT, 16)
        for k in range(ZPT // 16):
            pltpu.sync_copy(cp_v.at[pl.ds(0, 16)],
                            acc.at[pl.ds(zslab + k * 16, 16)])

        def part(i, cnt):
            dv = dst_v[pl.ds(i * 16, 16)]
            m = (dv >= lo) & (dv < lo + RANGE)
            mi = m.astype(jnp.int32)
            pos = cnt + plsc.cumsum(mi) - 1
            ri = lax.shift_right_logical(pos, 6)
            ci = pos & (CH - 1)
            plsc.store_scatter(eid_v, [ri, ci], sbase + i * 16 + i16, mask=m)
            return cnt + jnp.sum(mi)

        cnt = lax.fori_loop(0, ESL // 16, part, jnp.int32(0))
        nch = lax.shift_right_logical(cnt + (CH - 1), 6)
        cnt_pad = nch * CH
        sent = jnp.full((16,), sbase + ESL, jnp.int32)
        for k in range(CH // 16):
            pos = cnt + k * 16 + i16
            m = pos < cnt_pad
            ri = lax.shift_right_logical(pos, 6)
            ci = pos & (CH - 1)
            plsc.store_scatter(eid_v, [ri, ci], sent, mask=m)
        plsc.subcore_barrier()

        def chunk(j, carry):
            for k in range(CH // 16):
                eid = eid_v[j, pl.ds(k * 16, 16)]
                el = eid - sbase
                valid = el < ESL
                elc = jnp.minimum(el, ESL)
                dv = plsc.load_gather(dst_v, [elc])
                sv = plsc.load_gather(src_v, [elc])
                std_v[0, pl.ds(k * 16, 16)] = jnp.where(valid, dv - lo, RANGE)
                if with_c:
                    stab_v[0, pl.ds(k * 16, 16)] = jnp.where(valid, dv, 0)
                    stab_v[0, pl.ds(CH + k * 16, 16)] = (
                        jnp.where(valid, sv, 0) + NPA)
                    ste_v[0, pl.ds(k * 16, 16)] = jnp.where(valid, eid, 0)
                else:
                    sts_v[0, pl.ds(k * 16, 16)] = jnp.where(valid, sv, 0)
            if with_c:
                pltpu.sync_copy(ab_hbm.at[stab_v.at[0]], ab_v)
                pltpu.sync_copy(c_hbm.at[ste_v.at[0]], c_v)

                def row(r, rc):
                    for v in range(NV):
                        s = pl.ds(v * 16, 16)
                        ab_v[r, s] = jnp.maximum(
                            ab_v[r, s] + ab_v[CH + r, s] + c_v[r, s], 0.0)
                    return rc

                lax.fori_loop(0, CH, row, None)
                pltpu.sync_copy(ab_v.at[pl.ds(0, CH)], acc.at[std_v.at[0]],
                                add=True)
            else:
                pltpu.sync_copy(b_hbm.at[sts_v.at[0]], m_v)
                pltpu.sync_copy(m_v, acc.at[std_v.at[0]], add=True)
            return carry

        lax.fori_loop(0, nch, chunk, None)
        plsc.subcore_barrier()

        gbase = pl.multiple_of(cid * HALF, HALF) + p * RANGE
        oslab = pl.multiple_of(sid * OPT, OCH)
        for k in range(OPT // OCH):
            r0 = oslab + k * OCH
            pltpu.sync_copy(acc.at[pl.ds(r0, OCH)], cp_v)
            pltpu.sync_copy(cp_v, out_hbm.at[pl.ds(gbase + r0, OCH)])
        plsc.subcore_barrier()


def _deg_body(dst_hbm, deg_hbm, dst_v, deg_t):
    cid = lax.axis_index("c")
    sid = lax.axis_index("s")
    sbase = pl.multiple_of(sid * ESL, ESL)
    pltpu.sync_copy(dst_hbm.at[pl.ds(sbase, ESL)], dst_v.at[pl.ds(0, ESL)])
    zf = jnp.zeros((16,), jnp.float32)

    def zdeg(i, carry):
        deg_t[pl.ds(i * 16, 16)] = zf
        return carry

    lax.fori_loop(0, NP // 16, zdeg, None)
    onesf = jnp.ones((16,), jnp.float32)
    dbase = pl.multiple_of(cid * (ESL // 2), ESL // 2)

    def dloop(i, carry):
        dv = dst_v[pl.ds(dbase + i * 16, 16)]
        plsc.addupdate_scatter(deg_t, [dv], onesf)
        return carry

    lax.fori_loop(0, ESL // 2 // 16, dloop, None)
    wid = sid * NC + cid
    pltpu.sync_copy(deg_t, deg_hbm.at[pl.ds(pl.multiple_of(wid * NP, NP), NP)])


_deg_count = pl.kernel(
    _deg_body,
    out_type=(jax.ShapeDtypeStruct((NW * NP,), jnp.float32),),
    mesh=_MESH,
    compiler_params=pltpu.CompilerParams(needs_layout_passes=False),
    scratch_types=[
        pltpu.VMEM((ESL + 16,), jnp.int32),
        pltpu.VMEM((NP,), jnp.float32),
    ],
)


_COMMON_SCRATCH = [
    pltpu.VMEM((ESL + 16,), jnp.int32),
    pltpu.VMEM((ESL + 16,), jnp.int32),
    pltpu.VMEM((ERB, CH), jnp.int32),
    pltpu.VMEM((1, CH), jnp.int32),
]

_edge_scatter = pl.kernel(
    functools.partial(_scatter_body, "edge"),
    out_type=(jax.ShapeDtypeStruct((NP, H), jnp.float32),),
    mesh=_MESH,
    compiler_params=pltpu.CompilerParams(needs_layout_passes=False),
    scratch_types=_COMMON_SCRATCH + [
        pltpu.VMEM((1, 2 * CH), jnp.int32),
        pltpu.VMEM((1, CH), jnp.int32),
        pltpu.VMEM((2 * CH, H), jnp.float32),
        pltpu.VMEM((CH, H), jnp.float32),
        pltpu.VMEM((OCH, H), jnp.float32),
        pltpu.VMEM_SHARED((ACCR, H), jnp.float32),
    ],
)

_gcn_scatter = pl.kernel(
    functools.partial(_scatter_body, "gcn"),
    out_type=(jax.ShapeDtypeStruct((NP, H), jnp.float32),),
    mesh=_MESH,
    compiler_params=pltpu.CompilerParams(needs_layout_passes=False),
    scratch_types=_COMMON_SCRATCH + [
        pltpu.VMEM((1, CH), jnp.int32),
        pltpu.VMEM((CH, H), jnp.float32),
        pltpu.VMEM((OCH, H), jnp.float32),
        pltpu.VMEM_SHARED((ACCR, H), jnp.float32),
    ],
)


def _proj_body(x_ref, wa_ref, wb_ref, ab_ref):
    xv = x_ref[...]
    ab_ref[:NPA] = jnp.dot(xv, wa_ref[...],
                           preferred_element_type=jnp.float32)
    ab_ref[NPA:] = jnp.dot(xv, wb_ref[...],
                           preferred_element_type=jnp.float32)


_proj = pl.pallas_call(
    _proj_body,
    out_shape=jax.ShapeDtypeStruct((2 * NPA, H), jnp.float32),
)

_BE = 2000


def _cfeat_body(ea_ref, w1e_ref, b1_ref, w2e_ref, b2_ref, c1_ref, c2_ref):
    eav = ea_ref[...]
    c1_ref[...] = jnp.dot(eav, w1e_ref[...],
                          preferred_element_type=jnp.float32) + b1_ref[...]
    c2_ref[...] = jnp.dot(eav, w2e_ref[...],
                          preferred_element_type=jnp.float32) + b2_ref[...]


_cfeat = pl.pallas_call(
    _cfeat_body,
    grid=(E // _BE,),
    in_specs=[
        pl.BlockSpec((_BE, DE), lambda i: (i, 0)),
        pl.BlockSpec((DE, H), lambda i: (0, 0)),
        pl.BlockSpec((1, H), lambda i: (0, 0)),
        pl.BlockSpec((DE, H), lambda i: (0, 0)),
        pl.BlockSpec((1, H), lambda i: (0, 0)),
    ],
    out_specs=(pl.BlockSpec((_BE, H), lambda i: (i, 0)),
               pl.BlockSpec((_BE, H), lambda i: (i, 0))),
    out_shape=(jax.ShapeDtypeStruct((E, H), jnp.float32),
               jax.ShapeDtypeStruct((E, H), jnp.float32)),
)


def _bn_relu(h, g_ref, beta_ref):
    m = jnp.mean(h, axis=0, keepdims=True)
    v = jnp.mean((h - m) ** 2, axis=0, keepdims=True)
    hn = (h - m) * lax.rsqrt(v + 1e-5) * g_ref[...] + beta_ref[...]
    return jnp.maximum(hn, 0.0)


def _mid1_body(hp_ref, g1_ref, beta1_ref, w2a_ref, w2b_ref, ab2_ref):
    h = hp_ref[:N]
    hn = _bn_relu(h, g1_ref, beta1_ref)
    zpad = jnp.zeros((NPA - N, H), jnp.float32)
    ab2_ref[:N] = jnp.dot(hn, w2a_ref[...],
                          preferred_element_type=jnp.float32)
    ab2_ref[N:NPA] = zpad
    ab2_ref[NPA:NPA + N] = jnp.dot(hn, w2b_ref[...],
                                   preferred_element_type=jnp.float32)
    ab2_ref[NPA + N:] = zpad


_mid1 = pl.pallas_call(
    _mid1_body,
    out_shape=jax.ShapeDtypeStruct((2 * NPA, H), jnp.float32),
)


def _dinv_from(degp_ref):
    deg = jnp.sum(degp_ref[:, :N], axis=0) + 1.0
    return lax.rsqrt(deg)[:, None]


def _mid2_body(hp_ref, g2_ref, beta2_ref, wg_ref, degp_ref, gd_ref):
    h = hp_ref[:N]
    hn = _bn_relu(h, g2_ref, beta2_ref)
    g = jnp.dot(hn, wg_ref[...], preferred_element_type=jnp.float32)
    gd_ref[...] = g * _dinv_from(degp_ref)


_mid2 = pl.pallas_call(
    _mid2_body,
    out_shape=jax.ShapeDtypeStruct((N, H), jnp.float32),
)


def _final_body(sp_ref, gd_ref, degp_ref, bg_ref, wf_ref, bf_ref, out_ref):
    s = sp_ref[:N] + gd_ref[...]
    hg = s * _dinv_from(degp_ref) + bg_ref[...]
    out_ref[...] = jnp.dot(hg, wf_ref[...],
                           preferred_element_type=jnp.float32) + bf_ref[...]


_final = pl.pallas_call(
    _final_body,
    out_shape=jax.ShapeDtypeStruct((N, OUT), jnp.float32),
)


def kernel(x, edge_index, edge_attr, W1, b1, g1, beta1, W2, b2, g2, beta2,
           Wg, bg, Wf, bf):
    ei = edge_index.astype(jnp.int32)
    src = ei[0]
    dst = ei[1]

    x_pad = jnp.concatenate([x, jnp.zeros((NPA - N, D), jnp.float32)])
    ab1 = _proj(x_pad, W1[:D], W1[D:2 * D])
    c1, c2 = _cfeat(edge_attr, W1[2 * D:], b1.reshape(1, H),
                    W2[2 * H:], b2.reshape(1, H))

    (hp,) = _edge_scatter(ab1, c1, dst, src)
    (degf,) = _deg_count(dst)
    degp = degf.reshape(NW, NP)
    ab2 = _mid1(hp, g1.reshape(1, H), beta1.reshape(1, H),
                W2[:H], W2[H:2 * H])
    (hp2,) = _edge_scatter(ab2, c2, dst, src)
    gd = _mid2(hp2, g2.reshape(1, H), beta2.reshape(1, H), Wg, degp)
    (sp,) = _gcn_scatter(gd, dst, src)
    return _final(sp, gd, degp, bg.reshape(1, H), Wf, bf.reshape(1, OUT))
